# baseline (speedup 1.0000x reference)
import functools

import jax
import jax.numpy as jnp
from jax import lax
from jax.experimental import pallas as pl
from jax.experimental.pallas import tpu as pltpu
from jax.experimental.pallas import tpu_sc as plsc

N = 100000
V = 100
D = 128

R_TC = 2048
N_TC = 20 * R_TC
N_SC = N - N_TC

NC = 2
NS = 16
NW = NC * NS
ROWS_PER_W = N_SC // NW
S = 1920
CB = 128
N_CHUNKS = S // CB
NBUF = 6


def _sc_body(idx_hbm, table_hbm, out_hbm, idx_v, table_s, *rest):
    bufs = rest[:NBUF]
    sem_g = rest[NBUF:2 * NBUF]
    sem_s = rest[2 * NBUF:]

    wid = lax.axis_index("s") * NC + lax.axis_index("c")
    base = jnp.minimum((wid * ROWS_PER_W) // 8 * 8, N_SC - S)

    pltpu.sync_copy(table_hbm, table_s)
    pltpu.sync_copy(idx_hbm.at[pl.ds(N_TC + base, S)], idx_v)

    gath = {}
    scat = {}

    def start_gather(j):
        b = j % NBUF
        idx_ref = idx_v.at[pl.ds(j * CB, CB)]
        gath[j] = pltpu.async_copy(table_s.at[idx_ref], bufs[b], sem_g[b])

    for j in range(NBUF):
        start_gather(j)
    for j in range(N_CHUNKS):
        b = j % NBUF
        h = j + 1
        if NBUF <= h < N_CHUNKS:
            scat[h - NBUF].wait()
            start_gather(h)
        gath[j].wait()
        scat[j] = pltpu.async_copy(
            bufs[b], out_hbm.at[pl.ds(base + j * CB, CB)], sem_s[b])
    for j in range(N_CHUNKS - NBUF, N_CHUNKS):
        scat[j].wait()


@functools.partial(
    pl.kernel,
    mesh=plsc.VectorSubcoreMesh(core_axis_name="c", subcore_axis_name="s"),
    out_type=jax.ShapeDtypeStruct((N_SC, D), jnp.float32),
    scratch_types=[pltpu.VMEM((S,), jnp.int32),
                   pltpu.VMEM_SHARED((V, D), jnp.float32)]
    + [pltpu.VMEM((CB, D), jnp.float32) for _ in range(NBUF)]
    + [pltpu.SemaphoreType.DMA for _ in range(2 * NBUF)],
)
def _sc_gather(idx_hbm, table_hbm, out_hbm, idx_v, table_s, *rest):
    _sc_body(idx_hbm, table_hbm, out_hbm, idx_v, table_s, *rest)


def _tc_body(idx_ref, table_ref, out_ref):
    idxv = idx_ref[...]
    onehot = (idxv[:, None]
              == lax.broadcasted_iota(jnp.int32, (R_TC, V), 1)
              ).astype(jnp.float32)
    out_ref[...] = jnp.dot(onehot, table_ref[...],
                           preferred_element_type=jnp.float32)


_tc_gather = pl.pallas_call(
    _tc_body,
    grid=(N_TC // R_TC,),
    in_specs=[pl.BlockSpec((R_TC,), lambda i: (i,)),
              pl.BlockSpec((V, D), lambda i: (0, 0))],
    out_specs=pl.BlockSpec((R_TC, D), lambda i: (i, 0)),
    out_shape=jax.ShapeDtypeStruct((N_TC, D), jnp.float32),
)


def kernel(atomic_nums, embed_table):
    idx = atomic_nums.astype(jnp.int32)
    tc_part = _tc_gather(idx, embed_table)
    sc_part = _sc_gather(idx, embed_table)
    return jnp.concatenate([tc_part, sc_part], axis=0)


# speedup vs baseline: 1.6008x; 1.6008x over previous
import functools

import jax
import jax.numpy as jnp
from jax import lax
from jax.experimental import pallas as pl
from jax.experimental.pallas import tpu as pltpu
from jax.experimental.pallas import tpu_sc as plsc

N = 100000
D = 128
NC = 2
NS = 16
NW = NC * NS
ROWS_PER_W = 3125
S = 3200
CB = 128
N_CHUNKS = S // CB
NBUF = 6


def _body(idx_hbm, table_hbm, out_hbm, idx_v, table_v, *rest):
    bufs = rest[:NBUF]
    sem_g = rest[NBUF:2 * NBUF]
    sem_s = rest[2 * NBUF:]

    wid = lax.axis_index("s") * NC + lax.axis_index("c")
    base = jnp.minimum((wid * ROWS_PER_W) // 8 * 8, N - S)

    tcp = pltpu.async_copy(table_hbm, table_v, sem_g[0])
    icp = pltpu.async_copy(idx_hbm.at[pl.ds(base, S)], idx_v, sem_g[1])
    tcp.wait()
    icp.wait()

    gath = {}
    scat = {}

    def start_gather(j):
        b = j % NBUF
        idx_ref = idx_v.at[pl.ds(j * CB, CB)]
        gath[j] = pltpu.async_copy(table_v.at[idx_ref], bufs[b], sem_g[b])

    for j in range(NBUF):
        start_gather(j)
    for j in range(N_CHUNKS):
        b = j % NBUF
        h = j + 1
        if NBUF <= h < N_CHUNKS:
            scat[h - NBUF].wait()
            start_gather(h)
        gath[j].wait()
        scat[j] = pltpu.async_copy(
            bufs[b], out_hbm.at[pl.ds(base + j * CB, CB)], sem_s[b])
    for j in range(N_CHUNKS - NBUF, N_CHUNKS):
        scat[j].wait()


@functools.partial(
    pl.kernel,
    mesh=plsc.VectorSubcoreMesh(core_axis_name="c", subcore_axis_name="s"),
    out_type=jax.ShapeDtypeStruct((N, D), jnp.float32),
    scratch_types=[pltpu.VMEM((S,), jnp.int32),
                   pltpu.VMEM_SHARED((100, D), jnp.float32)]
    + [pltpu.VMEM((CB, D), jnp.float32) for _ in range(NBUF)]
    + [pltpu.SemaphoreType.DMA for _ in range(2 * NBUF)],
)
def _embed_gather(idx_hbm, table_hbm, out_hbm, idx_v, table_v, *rest):
    _body(idx_hbm, table_hbm, out_hbm, idx_v, table_v, *rest)


def kernel(atomic_nums, embed_table):
    return _embed_gather(atomic_nums.astype(jnp.int32), embed_table)


# speedup vs baseline: 1.6136x; 1.0080x over previous
import functools

import jax
import jax.numpy as jnp
from jax import lax
from jax.experimental import pallas as pl
from jax.experimental.pallas import tpu as pltpu
from jax.experimental.pallas import tpu_sc as plsc

N = 100000
D = 128
NC = 2
NS = 16
NW = NC * NS
ROWS_PER_W = 3125
S = 3200
CB = 128
N_CHUNKS = S // CB
NBUF = 7


def _body(idx_hbm, table_hbm, out_hbm, idx_v, table_v, *rest):
    bufs = rest[:NBUF]
    sem_g = rest[NBUF:2 * NBUF]
    sem_s = rest[2 * NBUF:]

    wid = lax.axis_index("s") * NC + lax.axis_index("c")
    base = jnp.minimum((wid * ROWS_PER_W) // 8 * 8, N - S)

    tcp = pltpu.async_copy(table_hbm, table_v, sem_g[0])
    icp = pltpu.async_copy(idx_hbm.at[pl.ds(base, S)], idx_v, sem_g[1])
    tcp.wait()
    icp.wait()

    gath = {}
    scat = {}

    def start_gather(j):
        b = j % NBUF
        idx_ref = idx_v.at[pl.ds(j * CB, CB)]
        gath[j] = pltpu.async_copy(table_v.at[idx_ref], bufs[b], sem_g[b])

    for j in range(NBUF):
        start_gather(j)
    for j in range(N_CHUNKS):
        b = j % NBUF
        h = j + 1
        if NBUF <= h < N_CHUNKS:
            scat[h - NBUF].wait()
            start_gather(h)
        gath[j].wait()
        scat[j] = pltpu.async_copy(
            bufs[b], out_hbm.at[pl.ds(base + j * CB, CB)], sem_s[b])
    for j in range(N_CHUNKS - NBUF, N_CHUNKS):
        scat[j].wait()


@functools.partial(
    pl.kernel,
    mesh=plsc.VectorSubcoreMesh(core_axis_name="c", subcore_axis_name="s"),
    out_type=jax.ShapeDtypeStruct((N, D), jnp.float32),
    scratch_types=[pltpu.VMEM((S,), jnp.int32),
                   pltpu.VMEM_SHARED((100, D), jnp.float32)]
    + [pltpu.VMEM((CB, D), jnp.float32) for _ in range(NBUF)]
    + [pltpu.SemaphoreType.DMA for _ in range(2 * NBUF)],
)
def _embed_gather(idx_hbm, table_hbm, out_hbm, idx_v, table_v, *rest):
    _body(idx_hbm, table_hbm, out_hbm, idx_v, table_v, *rest)


def kernel(atomic_nums, embed_table):
    return _embed_gather(atomic_nums.astype(jnp.int32), embed_table)


# speedup vs baseline: 1.6957x; 1.0509x over previous
import functools

import jax
import jax.numpy as jnp
from jax import lax
from jax.experimental import pallas as pl
from jax.experimental.pallas import tpu as pltpu
from jax.experimental.pallas import tpu_sc as plsc

N = 100000
D = 128
NC = 2
NS = 16
NW = NC * NS
ROWS_PER_W = 3125
S = 3200
CB = 128
N_CHUNKS = S // CB
NBUF = 7


def _body(idx_hbm, table_hbm, out_hbm, idx_v, table_v, *rest):
    bufs = rest[:NBUF]
    sem_g = rest[NBUF:2 * NBUF]
    sem_s = rest[2 * NBUF:]

    wid = lax.axis_index("s") * NC + lax.axis_index("c")
    base = jnp.minimum((wid * ROWS_PER_W) // 8 * 8, N - S)

    icp = pltpu.async_copy(idx_hbm.at[pl.ds(base, S)], idx_v, sem_g[1])

    @pl.when(lax.axis_index("s") == 0)
    def _copy_table():
        pltpu.async_copy(table_hbm, table_v, sem_g[0]).wait()

    plsc.subcore_barrier()
    icp.wait()

    gath = {}
    scat = {}

    def start_gather(j):
        b = j % NBUF
        idx_ref = idx_v.at[pl.ds(j * CB, CB)]
        gath[j] = pltpu.async_copy(table_v.at[idx_ref], bufs[b], sem_g[b])

    for j in range(NBUF):
        start_gather(j)
    for j in range(N_CHUNKS):
        b = j % NBUF
        h = j + 1
        if NBUF <= h < N_CHUNKS:
            scat[h - NBUF].wait()
            start_gather(h)
        gath[j].wait()
        scat[j] = pltpu.async_copy(
            bufs[b], out_hbm.at[pl.ds(base + j * CB, CB)], sem_s[b])
    for j in range(N_CHUNKS - NBUF, N_CHUNKS):
        scat[j].wait()


@functools.partial(
    pl.kernel,
    mesh=plsc.VectorSubcoreMesh(core_axis_name="c", subcore_axis_name="s"),
    out_type=jax.ShapeDtypeStruct((N, D), jnp.float32),
    scratch_types=[pltpu.VMEM((S,), jnp.int32),
                   pltpu.VMEM_SHARED((100, D), jnp.float32)]
    + [pltpu.VMEM((CB, D), jnp.float32) for _ in range(NBUF)]
    + [pltpu.SemaphoreType.DMA for _ in range(2 * NBUF)],
)
def _embed_gather(idx_hbm, table_hbm, out_hbm, idx_v, table_v, *rest):
    _body(idx_hbm, table_hbm, out_hbm, idx_v, table_v, *rest)


def kernel(atomic_nums, embed_table):
    return _embed_gather(atomic_nums.astype(jnp.int32), embed_table)


# speedup vs baseline: 1.7060x; 1.0061x over previous
import functools

import jax
import jax.numpy as jnp
from jax import lax
from jax.experimental import pallas as pl
from jax.experimental.pallas import tpu as pltpu
from jax.experimental.pallas import tpu_sc as plsc

N = 100000
D = 128
NC = 2
NS = 16
NW = NC * NS
ROWS_PER_W = 3125
S = 3136
CB = 128
SIZES = [CB] * 24 + [64]
OFFS = [CB * j for j in range(25)]
N_CHUNKS = len(SIZES)
NBUF = 7


def _body(idx_hbm, table_hbm, out_hbm, idx_v, table_v, *rest):
    bufs = rest[:NBUF]
    sem_g = rest[NBUF:2 * NBUF]
    sem_s = rest[2 * NBUF:]

    wid = lax.axis_index("s") * NC + lax.axis_index("c")
    base = jnp.minimum((wid * ROWS_PER_W) // 8 * 8, N - S)

    icp = pltpu.async_copy(idx_hbm.at[pl.ds(base, S)], idx_v, sem_g[1])

    @pl.when(lax.axis_index("s") == 0)
    def _copy_table():
        pltpu.async_copy(table_hbm, table_v, sem_g[0]).wait()

    plsc.subcore_barrier()
    icp.wait()

    gath = {}
    scat = {}

    def buf_dst(j):
        b = j % NBUF
        sz = SIZES[j]
        return bufs[b] if sz == CB else bufs[b].at[pl.ds(0, sz)]

    def start_gather(j):
        b = j % NBUF
        idx_ref = idx_v.at[pl.ds(OFFS[j], SIZES[j])]
        gath[j] = pltpu.async_copy(table_v.at[idx_ref], buf_dst(j), sem_g[b])

    for j in range(NBUF):
        start_gather(j)
    for j in range(N_CHUNKS):
        b = j % NBUF
        h = j + 1
        if NBUF <= h < N_CHUNKS:
            scat[h - NBUF].wait()
            start_gather(h)
        gath[j].wait()
        scat[j] = pltpu.async_copy(
            buf_dst(j), out_hbm.at[pl.ds(base + OFFS[j], SIZES[j])], sem_s[b])
    for j in range(N_CHUNKS - NBUF, N_CHUNKS):
        scat[j].wait()


@functools.partial(
    pl.kernel,
    mesh=plsc.VectorSubcoreMesh(core_axis_name="c", subcore_axis_name="s"),
    out_type=jax.ShapeDtypeStruct((N, D), jnp.float32),
    scratch_types=[pltpu.VMEM((S,), jnp.int32),
                   pltpu.VMEM_SHARED((100, D), jnp.float32)]
    + [pltpu.VMEM((CB, D), jnp.float32) for _ in range(NBUF)]
    + [pltpu.SemaphoreType.DMA for _ in range(2 * NBUF)],
)
def _embed_gather(idx_hbm, table_hbm, out_hbm, idx_v, table_v, *rest):
    _body(idx_hbm, table_hbm, out_hbm, idx_v, table_v, *rest)


def kernel(atomic_nums, embed_table):
    return _embed_gather(atomic_nums.astype(jnp.int32), embed_table)
